# Initial kernel scaffold; baseline (speedup 1.0000x reference)
import jax
import jax.numpy as jnp
from jax.experimental import pallas as pl


def kernel(K_curr, V_curr, K_prev, V_prev, h, momentum, active_idx, Wk, bk, Wv, bv, logit_eta, surprise_logit_alpha):
    raise NotImplementedError("write your pallas kernel here")


# speedup vs baseline: 8.4639x; 8.4639x over previous
import functools

import jax
import jax.numpy as jnp
from jax.experimental import pallas as pl
from jax.experimental.pallas import tpu as pltpu


_HI = jax.lax.Precision.HIGHEST


def _qprobe_body(h_ref, o_ref, *, nsteps, seq):
    s = pl.program_id(1)

    @pl.when(s == 0)
    def _():
        o_ref[...] = jnp.zeros_like(o_ref)

    o_ref[...] += jnp.sum(h_ref[...], axis=1, keepdims=True) * (1.0 / seq)


def _qprobe(h, cs=512):
    b, seq, d = h.shape
    grid = (b, seq // cs)
    return pl.pallas_call(
        functools.partial(_qprobe_body, nsteps=seq // cs, seq=seq),
        grid=grid,
        in_specs=[pl.BlockSpec((1, cs, d), lambda i, s: (i, s, 0))],
        out_specs=pl.BlockSpec((1, 1, d), lambda i, s: (i, 0, 0)),
        out_shape=jax.ShapeDtypeStruct((b, 1, d), jnp.float32),
        compiler_params=pltpu.CompilerParams(
            dimension_semantics=("parallel", "arbitrary")),
    )(h)


def _logits_body(idx_ref, q_ref, kc_ref, vc_ref, kl_ref, vl_ref, *, cm, scale):
    mc = pl.program_id(1)

    @pl.when(mc == 0)
    def _():
        kl_ref[...] = jnp.zeros_like(kl_ref)
        vl_ref[...] = jnp.zeros_like(vl_ref)

    q = q_ref[0]
    idx = idx_ref[0]
    m_ids = mc * cm + jax.lax.broadcasted_iota(jnp.int32, (cm, 1), 0)
    sel = (idx == m_ids)

    for (c_ref, l_ref) in ((kc_ref, kl_ref), (vc_ref, vl_ref)):
        dots = jax.lax.dot_general(
            c_ref[0], q, (((1,), (1,)), ((), ())),
            precision=_HI) * scale
        l_ref[0] += jnp.sum(jnp.where(sel, dots, 0.0), axis=0,
                            keepdims=True)


def _logits(idx3, q3, K_curr, V_curr, cm=256):
    b, m, d = K_curr.shape
    ng = idx3.shape[-1]
    scale = float(d) ** -0.5
    grid = (b, m // cm)
    small = jax.ShapeDtypeStruct((b, 1, ng), jnp.float32)
    return pl.pallas_call(
        functools.partial(_logits_body, cm=cm, scale=scale),
        grid=grid,
        in_specs=[
            pl.BlockSpec((1, 1, ng), lambda i, mc: (i, 0, 0)),
            pl.BlockSpec((1, 1, d), lambda i, mc: (i, 0, 0)),
            pl.BlockSpec((1, cm, d), lambda i, mc: (i, mc, 0)),
            pl.BlockSpec((1, cm, d), lambda i, mc: (i, mc, 0)),
        ],
        out_specs=[
            pl.BlockSpec((1, 1, ng), lambda i, mc: (i, 0, 0)),
            pl.BlockSpec((1, 1, ng), lambda i, mc: (i, 0, 0)),
        ],
        out_shape=[small, small],
        compiler_params=pltpu.CompilerParams(
            dimension_semantics=("parallel", "arbitrary")),
    )(idx3, q3, K_curr, V_curr)


def _softmax_body(kl_ref, vl_ref, ka_ref, va_ref):
    for (l_ref, a_ref) in ((kl_ref, ka_ref), (vl_ref, va_ref)):
        x = l_ref[...]
        x = x - jnp.max(x, axis=-1, keepdims=True)
        e = jnp.exp(x)
        a_ref[...] = e / jnp.sum(e, axis=-1, keepdims=True)


def _softmax(kl, vl):
    shp = jax.ShapeDtypeStruct(kl.shape, jnp.float32)
    return pl.pallas_call(
        _softmax_body,
        out_shape=[shp, shp],
    )(kl, vl)


def _pred_body(idx_ref, ka_ref, va_ref, kc_ref, vc_ref, kp_ref, vp_ref, *, cm):
    mc = pl.program_id(1)

    @pl.when(mc == 0)
    def _():
        kp_ref[...] = jnp.zeros_like(kp_ref)
        vp_ref[...] = jnp.zeros_like(vp_ref)

    idx = idx_ref[0]
    m_ids = mc * cm + jax.lax.broadcasted_iota(jnp.int32, (cm, 1), 0)
    sel = (idx == m_ids)

    for (a_ref, c_ref, p_ref) in ((ka_ref, kc_ref, kp_ref),
                                  (va_ref, vc_ref, vp_ref)):
        w = jnp.sum(jnp.where(sel, a_ref[0], 0.0), axis=1,
                    keepdims=True)
        p_ref[0] += jax.lax.dot_general(
            w, c_ref[0], (((0,), (0,)), ((), ())), precision=_HI)


def _predicted(idx3, ka, va, K_curr, V_curr, cm=256):
    b, m, d = K_curr.shape
    ng = idx3.shape[-1]
    grid = (b, m // cm)
    out = jax.ShapeDtypeStruct((b, 1, d), jnp.float32)
    return pl.pallas_call(
        functools.partial(_pred_body, cm=cm),
        grid=grid,
        in_specs=[
            pl.BlockSpec((1, 1, ng), lambda i, mc: (i, 0, 0)),
            pl.BlockSpec((1, 1, ng), lambda i, mc: (i, 0, 0)),
            pl.BlockSpec((1, 1, ng), lambda i, mc: (i, 0, 0)),
            pl.BlockSpec((1, cm, d), lambda i, mc: (i, mc, 0)),
            pl.BlockSpec((1, cm, d), lambda i, mc: (i, mc, 0)),
        ],
        out_specs=[
            pl.BlockSpec((1, 1, d), lambda i, mc: (i, 0, 0)),
            pl.BlockSpec((1, 1, d), lambda i, mc: (i, 0, 0)),
        ],
        out_shape=[out, out],
        compiler_params=pltpu.CompilerParams(
            dimension_semantics=("parallel", "arbitrary")),
    )(idx3, ka, va, K_curr, V_curr)


def _stats_body(kp_ref, vp_ref, q_ref, mom_ref, wk_ref, bk_ref, wv_ref,
                bv_ref, eta_ref, alpha_ref, nm_ref, kg_ref, vg_ref):
    q = q_ref[:, 0, :]
    kp = kp_ref[:, 0, :]
    vp = vp_ref[:, 0, :]
    ks = jnp.mean((kp - q) ** 2, axis=-1, keepdims=True)
    vs = jnp.mean((vp - q) ** 2, axis=-1, keepdims=True)
    alpha = jax.nn.sigmoid(alpha_ref[0, 0])
    comb = alpha * ks + (1.0 - alpha) * vs
    eta = jax.nn.sigmoid(eta_ref[0, 0])
    nm = eta * mom_ref[...] + (1.0 - eta) * comb
    nm_ref[...] = nm
    for (s, w_ref, b_ref, g_ref) in ((ks, wk_ref, bk_ref, kg_ref),
                                     (vs, wv_ref, bv_ref, vg_ref)):
        z = (s * w_ref[:, 0:1].T + nm * w_ref[:, 1:2].T + b_ref[...])
        g_ref[...] = jax.nn.sigmoid(z)[:, None, :]


def _stats(kp, vp, q3, momentum, Wk, bk2, Wv, bv2, eta2, alpha2):
    b = kp.shape[0]
    ng = Wk.shape[0]
    return pl.pallas_call(
        _stats_body,
        out_shape=[
            jax.ShapeDtypeStruct((b, 1), jnp.float32),
            jax.ShapeDtypeStruct((b, 1, ng), jnp.float32),
            jax.ShapeDtypeStruct((b, 1, ng), jnp.float32),
        ],
    )(kp, vp, q3, momentum, Wk, bk2, Wv, bv2, eta2, alpha2)


def _merge_body(idx_ref, kg_ref, vg_ref, kc_ref, kp_ref, vc_ref, vp_ref,
                ko_ref, vo_ref, *, tm):
    mt = pl.program_id(1)
    idx = idx_ref[0]
    ng = idx.shape[-1]
    m_ids = mt * tm + jax.lax.broadcasted_iota(jnp.int32, (tm, 1), 0)
    sel = (idx == m_ids)
    n_iota = jax.lax.broadcasted_iota(jnp.int32, (tm, ng), 1)
    n_sel = jnp.max(jnp.where(sel, n_iota, -1), axis=1, keepdims=True)
    pick = sel & (n_iota == n_sel)
    active = n_sel >= 0
    for (g_ref, c_ref, p_ref, o_ref) in ((kg_ref, kc_ref, kp_ref, ko_ref),
                                         (vg_ref, vc_ref, vp_ref, vo_ref)):
        g = jnp.sum(jnp.where(pick, g_ref[0], 0.0), axis=1, keepdims=True)
        g = jnp.where(active, g, 1.0)
        o_ref[0] = g * c_ref[0] + (1.0 - g) * p_ref[0]


def _merge(idx3, kg, vg, K_curr, K_prev, V_curr, V_prev, tm=256):
    b, m, d = K_curr.shape
    ng = idx3.shape[-1]
    grid = (b, m // tm)
    big = pl.BlockSpec((1, tm, d), lambda i, mt: (i, mt, 0))
    small = pl.BlockSpec((1, 1, ng), lambda i, mt: (i, 0, 0))
    out = jax.ShapeDtypeStruct((b, m, d), jnp.float32)
    return pl.pallas_call(
        functools.partial(_merge_body, tm=tm),
        grid=grid,
        in_specs=[small, small, small, big, big, big, big],
        out_specs=[big, big],
        out_shape=[out, out],
        compiler_params=pltpu.CompilerParams(
            dimension_semantics=("parallel", "arbitrary")),
    )(idx3, kg, vg, K_curr, K_prev, V_curr, V_prev)


def kernel(K_curr, V_curr, K_prev, V_prev, h, momentum, active_idx,
           Wk, bk, Wv, bv, logit_eta, surprise_logit_alpha):
    b, m, d = K_curr.shape
    ng = active_idx.shape[1]

    idx3 = active_idx.astype(jnp.int32).reshape(b, 1, ng)
    q3 = _qprobe(h)
    kl, vl = _logits(idx3, q3, K_curr, V_curr)
    ka, va = _softmax(kl, vl)
    kp, vp = _predicted(idx3, ka, va, K_curr, V_curr)
    nm, kg, vg = _stats(
        kp, vp, q3, momentum, Wk, bk.reshape(1, ng), Wv, bv.reshape(1, ng),
        jnp.reshape(logit_eta, (1, 1)),
        jnp.reshape(surprise_logit_alpha, (1, 1)))
    K_out, V_out = _merge(idx3, kg, vg, K_curr, K_prev, V_curr, V_prev)
    return (K_out, V_out, nm)


# speedup vs baseline: 8.7149x; 1.0297x over previous
import functools

import jax
import jax.numpy as jnp
from jax.experimental import pallas as pl
from jax.experimental.pallas import tpu as pltpu


_HI = jax.lax.Precision.HIGHEST
_NEG = -1e30


def _mult_body(idx_ref, mult_ref, *, cm):
    mc = pl.program_id(1)
    idx = idx_ref[0]
    m_ids = mc * cm + jax.lax.broadcasted_iota(jnp.int32, (cm, 1), 0)
    sel = (idx == m_ids)
    mult_ref[0] = jnp.sum(jnp.where(sel, 1.0, 0.0), axis=1, keepdims=True)


def _mult(idx3, m, cm=512):
    b, _, ng = idx3.shape
    return pl.pallas_call(
        functools.partial(_mult_body, cm=cm),
        grid=(b, m // cm),
        in_specs=[pl.BlockSpec((1, 1, ng), lambda i, mc: (i, 0, 0))],
        out_specs=pl.BlockSpec((1, cm, 1), lambda i, mc: (i, mc, 0)),
        out_shape=jax.ShapeDtypeStruct((b, m, 1), jnp.float32),
        compiler_params=pltpu.CompilerParams(
            dimension_semantics=("parallel", "parallel")),
    )(idx3)


def _qprobe_body(h_ref, o_ref, *, seq):
    s = pl.program_id(1)

    @pl.when(s == 0)
    def _():
        o_ref[...] = jnp.zeros_like(o_ref)

    o_ref[...] += jnp.sum(h_ref[...], axis=1, keepdims=True) * (1.0 / seq)


def _qprobe(h, cs=512):
    b, seq, d = h.shape
    return pl.pallas_call(
        functools.partial(_qprobe_body, seq=seq),
        grid=(b, seq // cs),
        in_specs=[pl.BlockSpec((1, cs, d), lambda i, s: (i, s, 0))],
        out_specs=pl.BlockSpec((1, 1, d), lambda i, s: (i, 0, 0)),
        out_shape=jax.ShapeDtypeStruct((b, 1, d), jnp.float32),
        compiler_params=pltpu.CompilerParams(
            dimension_semantics=("parallel", "arbitrary")),
    )(h)


def _attn_body(mult_ref, q_ref, kc_ref, vc_ref,
               ka_ref, kd_ref, km_ref, va_ref, vd_ref, vm_ref, *, cm, scale):
    mc = pl.program_id(1)

    @pl.when(mc == 0)
    def _():
        for r in (ka_ref, va_ref, kd_ref, vd_ref):
            r[...] = jnp.zeros_like(r)
        km_ref[...] = jnp.full_like(km_ref, _NEG)
        vm_ref[...] = jnp.full_like(vm_ref, _NEG)

    q = q_ref[0]
    mult = mult_ref[0]
    has = mult > 0.0

    for (c_ref, a_ref, d_ref, m_ref) in ((kc_ref, ka_ref, kd_ref, km_ref),
                                         (vc_ref, va_ref, vd_ref, vm_ref)):
        rows = c_ref[0]
        l = jax.lax.dot_general(rows, q, (((1,), (1,)), ((), ())),
                                precision=_HI) * scale
        lm = jnp.where(has, l, _NEG)
        cmax = jnp.max(lm)
        m_old = m_ref[...]
        m_new = jnp.maximum(m_old, cmax)
        factor = jnp.exp(m_old - m_new)
        e = jnp.exp(l - m_new[0]) * mult
        a_ref[...] = a_ref[...] * factor + jax.lax.dot_general(
            e, rows, (((0,), (0,)), ((), ())), precision=_HI)[None]
        d_ref[...] = d_ref[...] * factor + jnp.sum(e)
        m_ref[...] = m_new


def _attn(mult_f, q3, K_curr, V_curr, cm=256):
    b, m, d = K_curr.shape
    scale = float(d) ** -0.5
    vec = jax.ShapeDtypeStruct((b, 1, d), jnp.float32)
    sca = jax.ShapeDtypeStruct((b, 1, 1), jnp.float32)
    vspec = pl.BlockSpec((1, 1, d), lambda i, mc: (i, 0, 0))
    sspec = pl.BlockSpec((1, 1, 1), lambda i, mc: (i, 0, 0))
    big = pl.BlockSpec((1, cm, d), lambda i, mc: (i, mc, 0))
    return pl.pallas_call(
        functools.partial(_attn_body, cm=cm, scale=scale),
        grid=(b, m // cm),
        in_specs=[
            pl.BlockSpec((1, cm, 1), lambda i, mc: (i, mc, 0)),
            vspec, big, big,
        ],
        out_specs=[vspec, sspec, sspec, vspec, sspec, sspec],
        out_shape=[vec, sca, sca, vec, sca, sca],
        compiler_params=pltpu.CompilerParams(
            dimension_semantics=("parallel", "arbitrary")),
    )(mult_f, q3, K_curr, V_curr)


def _stats_body(ka_ref, kd_ref, va_ref, vd_ref, q_ref, mom_ref, wk_ref,
                bk_ref, wv_ref, bv_ref, eta_ref, alpha_ref,
                nm_ref, kg_ref, vg_ref):
    q = q_ref[:, 0, :]
    kp = ka_ref[:, 0, :] / kd_ref[:, 0, :]
    vp = va_ref[:, 0, :] / vd_ref[:, 0, :]
    ks = jnp.mean((kp - q) ** 2, axis=-1, keepdims=True)
    vs = jnp.mean((vp - q) ** 2, axis=-1, keepdims=True)
    alpha = jax.nn.sigmoid(alpha_ref[0, 0])
    comb = alpha * ks + (1.0 - alpha) * vs
    eta = jax.nn.sigmoid(eta_ref[0, 0])
    nm = eta * mom_ref[...] + (1.0 - eta) * comb
    nm_ref[...] = nm
    for (s, w_ref, b_ref, g_ref) in ((ks, wk_ref, bk_ref, kg_ref),
                                     (vs, wv_ref, bv_ref, vg_ref)):
        z = s * w_ref[:, 0:1].T + nm * w_ref[:, 1:2].T + b_ref[...]
        g_ref[...] = jax.nn.sigmoid(z)[:, None, :]


def _stats(ka, kd, va, vd, q3, momentum, Wk, bk2, Wv, bv2, eta2, alpha2):
    b = ka.shape[0]
    ng = Wk.shape[0]
    return pl.pallas_call(
        _stats_body,
        out_shape=[
            jax.ShapeDtypeStruct((b, 1), jnp.float32),
            jax.ShapeDtypeStruct((b, 1, ng), jnp.float32),
            jax.ShapeDtypeStruct((b, 1, ng), jnp.float32),
        ],
    )(ka, kd, va, vd, q3, momentum, Wk, bk2, Wv, bv2, eta2, alpha2)


def _gatefull_body(idx_ref, kg_ref, vg_ref, gk_ref, gv_ref, *, cm):
    mc = pl.program_id(1)
    idx = idx_ref[0]
    ng = idx.shape[-1]
    m_ids = mc * cm + jax.lax.broadcasted_iota(jnp.int32, (cm, 1), 0)
    sel = (idx == m_ids)
    n_iota = jax.lax.broadcasted_iota(jnp.int32, (cm, ng), 1)
    n_sel = jnp.max(jnp.where(sel, n_iota, -1), axis=1, keepdims=True)
    pick = sel & (n_iota == n_sel)
    active = n_sel >= 0
    for (g_ref, o_ref) in ((kg_ref, gk_ref), (vg_ref, gv_ref)):
        g = jnp.sum(jnp.where(pick, g_ref[0], 0.0), axis=1, keepdims=True)
        o_ref[0] = jnp.where(active, g, 1.0)


def _gatefull(idx3, kg, vg, m, cm=512):
    b, _, ng = idx3.shape
    small = pl.BlockSpec((1, 1, ng), lambda i, mc: (i, 0, 0))
    out = jax.ShapeDtypeStruct((b, m, 1), jnp.float32)
    ospec = pl.BlockSpec((1, cm, 1), lambda i, mc: (i, mc, 0))
    return pl.pallas_call(
        functools.partial(_gatefull_body, cm=cm),
        grid=(b, m // cm),
        in_specs=[small, small, small],
        out_specs=[ospec, ospec],
        out_shape=[out, out],
        compiler_params=pltpu.CompilerParams(
            dimension_semantics=("parallel", "parallel")),
    )(idx3, kg, vg)


def _merge_body(gk_ref, gv_ref, kc_ref, kp_ref, vc_ref, vp_ref,
                ko_ref, vo_ref):
    for (g_ref, c_ref, p_ref, o_ref) in ((gk_ref, kc_ref, kp_ref, ko_ref),
                                         (gv_ref, vc_ref, vp_ref, vo_ref)):
        g = g_ref[...]
        o_ref[...] = g * c_ref[...] + (1.0 - g) * p_ref[...]


def _merge(gk_f, gv_f, K_curr, K_prev, V_curr, V_prev, tm=256):
    b, m, d = K_curr.shape
    big = pl.BlockSpec((1, tm, d), lambda i, mt: (i, mt, 0))
    gsp = pl.BlockSpec((1, tm, 1), lambda i, mt: (i, mt, 0))
    out = jax.ShapeDtypeStruct((b, m, d), jnp.float32)
    return pl.pallas_call(
        _merge_body,
        grid=(b, m // tm),
        in_specs=[gsp, gsp, big, big, big, big],
        out_specs=[big, big],
        out_shape=[out, out],
        compiler_params=pltpu.CompilerParams(
            dimension_semantics=("parallel", "parallel")),
    )(gk_f, gv_f, K_curr, K_prev, V_curr, V_prev)


def kernel(K_curr, V_curr, K_prev, V_prev, h, momentum, active_idx,
           Wk, bk, Wv, bv, logit_eta, surprise_logit_alpha):
    b, m, d = K_curr.shape
    ng = active_idx.shape[1]

    idx3 = active_idx.astype(jnp.int32).reshape(b, 1, ng)
    mult_f = _mult(idx3, m)
    q3 = _qprobe(h)
    ka, kd, _, va, vd, _ = _attn(mult_f, q3, K_curr, V_curr)
    nm, kg, vg = _stats(
        ka, kd, va, vd, q3, momentum, Wk, bk.reshape(1, ng), Wv,
        bv.reshape(1, ng), jnp.reshape(logit_eta, (1, 1)),
        jnp.reshape(surprise_logit_alpha, (1, 1)))
    gk_f, gv_f = _gatefull(idx3, kg, vg, m)
    K_out, V_out = _merge(gk_f, gv_f, K_curr, K_prev, V_curr, V_prev)
    return (K_out, V_out, nm)


# speedup vs baseline: 9.7126x; 1.1145x over previous
import functools

import jax
import jax.numpy as jnp
from jax.experimental import pallas as pl
from jax.experimental.pallas import tpu as pltpu


_HI = jax.lax.Precision.HIGHEST
_NEG = -1e30


def _prep_body(h_ref, idx_ref, q_ref, mult_ref, *, cm):
    q_ref[...] = jnp.mean(h_ref[...], axis=1, keepdims=True)
    idx = idx_ref[0]
    m = mult_ref.shape[1]
    for c in range(m // cm):
        m_ids = c * cm + jax.lax.broadcasted_iota(jnp.int32, (cm, 1), 0)
        sel = (idx == m_ids)
        mult_ref[0, pl.ds(c * cm, cm), :] = jnp.sum(
            jnp.where(sel, 1.0, 0.0), axis=1, keepdims=True)


def _prep(h, idx3, m, cm=512):
    b, seq, d = h.shape
    ng = idx3.shape[-1]
    return pl.pallas_call(
        functools.partial(_prep_body, cm=cm),
        grid=(b,),
        in_specs=[
            pl.BlockSpec((1, seq, d), lambda i: (i, 0, 0)),
            pl.BlockSpec((1, 1, ng), lambda i: (i, 0, 0)),
        ],
        out_specs=[
            pl.BlockSpec((1, 1, d), lambda i: (i, 0, 0)),
            pl.BlockSpec((1, m, 1), lambda i: (i, 0, 0)),
        ],
        out_shape=[
            jax.ShapeDtypeStruct((b, 1, d), jnp.float32),
            jax.ShapeDtypeStruct((b, m, 1), jnp.float32),
        ],
        compiler_params=pltpu.CompilerParams(
            dimension_semantics=("arbitrary",)),
    )(h, idx3)


def _surprise_body(mult_ref, q_ref, kc_ref, vc_ref, ks_ref, vs_ref, *, scale):
    q = q_ref[0]
    mult = mult_ref[0]
    has = mult > 0.0
    for (c_ref, s_ref) in ((kc_ref, ks_ref), (vc_ref, vs_ref)):
        rows = c_ref[0]
        l = jax.lax.dot_general(rows, q, (((1,), (1,)), ((), ())),
                                precision=_HI) * scale
        lmax = jnp.max(jnp.where(has, l, _NEG))
        e = jnp.exp(l - lmax) * mult
        den = jnp.sum(e)
        num = jax.lax.dot_general(e, rows, (((0,), (0,)), ((), ())),
                                  precision=_HI)
        pred = num / den
        s_ref[...] = jnp.mean((pred - q) ** 2)[None, None, None]


def _surprise(mult_f, q3, K_curr, V_curr):
    b, m, d = K_curr.shape
    scale = float(d) ** -0.5
    sca = jax.ShapeDtypeStruct((b, 1, 1), jnp.float32)
    sspec = pl.BlockSpec((1, 1, 1), lambda i: (i, 0, 0))
    big = pl.BlockSpec((1, m, d), lambda i: (i, 0, 0))
    return pl.pallas_call(
        functools.partial(_surprise_body, scale=scale),
        grid=(b,),
        in_specs=[
            pl.BlockSpec((1, m, 1), lambda i: (i, 0, 0)),
            pl.BlockSpec((1, 1, d), lambda i: (i, 0, 0)),
            big, big,
        ],
        out_specs=[sspec, sspec],
        out_shape=[sca, sca],
        compiler_params=pltpu.CompilerParams(
            dimension_semantics=("arbitrary",)),
    )(mult_f, q3, K_curr, V_curr)


def _gates_body(ks_ref, vs_ref, mom_ref, idx_ref, wk_ref, bk_ref, wv_ref,
                bv_ref, eta_ref, alpha_ref, nm_ref, gk_ref, gv_ref, *, cm):
    ks = ks_ref[...]
    vs = vs_ref[...]
    alpha = jax.nn.sigmoid(alpha_ref[0, 0])
    comb = alpha * ks + (1.0 - alpha) * vs
    eta = jax.nn.sigmoid(eta_ref[0, 0])
    nm = eta * mom_ref[...] + (1.0 - eta) * comb
    nm_ref[...] = nm

    idx = idx_ref[0]
    ng = idx.shape[-1]
    m = gk_ref.shape[1]
    gk = jax.nn.sigmoid(ks[0] * wk_ref[:, 0:1].T + nm[0] * wk_ref[:, 1:2].T
                        + bk_ref[...])
    gv = jax.nn.sigmoid(vs[0] * wv_ref[:, 0:1].T + nm[0] * wv_ref[:, 1:2].T
                        + bv_ref[...])
    for c in range(m // cm):
        m_ids = c * cm + jax.lax.broadcasted_iota(jnp.int32, (cm, 1), 0)
        sel = (idx == m_ids)
        n_iota = jax.lax.broadcasted_iota(jnp.int32, (cm, ng), 1)
        n_sel = jnp.max(jnp.where(sel, n_iota, -1), axis=1, keepdims=True)
        pick = sel & (n_iota == n_sel)
        active = n_sel >= 0
        for (g, o_ref) in ((gk, gk_ref), (gv, gv_ref)):
            gc = jnp.sum(jnp.where(pick, g, 0.0), axis=1, keepdims=True)
            o_ref[0, pl.ds(c * cm, cm), :] = jnp.where(active, gc, 1.0)


def _gates(ks, vs, mom3, idx3, Wk, bk2, Wv, bv2, eta2, alpha2, m, cm=512):
    b, _, ng = idx3.shape
    sspec = pl.BlockSpec((1, 1, 1), lambda i: (i, 0, 0))
    gspec = pl.BlockSpec((1, m, 1), lambda i: (i, 0, 0))
    whole = lambda shape: pl.BlockSpec(shape, lambda i: tuple(0 for _ in shape))
    return pl.pallas_call(
        functools.partial(_gates_body, cm=cm),
        grid=(b,),
        in_specs=[
            sspec, sspec, sspec,
            pl.BlockSpec((1, 1, ng), lambda i: (i, 0, 0)),
            whole((ng, 2)), whole((1, ng)), whole((ng, 2)), whole((1, ng)),
            whole((1, 1)), whole((1, 1)),
        ],
        out_specs=[sspec, gspec, gspec],
        out_shape=[
            jax.ShapeDtypeStruct((b, 1, 1), jnp.float32),
            jax.ShapeDtypeStruct((b, m, 1), jnp.float32),
            jax.ShapeDtypeStruct((b, m, 1), jnp.float32),
        ],
        compiler_params=pltpu.CompilerParams(
            dimension_semantics=("arbitrary",)),
    )(ks, vs, mom3, idx3, Wk, bk2, Wv, bv2, eta2, alpha2)


def _merge_body(gk_ref, gv_ref, kc_ref, kp_ref, vc_ref, vp_ref,
                ko_ref, vo_ref):
    for (g_ref, c_ref, p_ref, o_ref) in ((gk_ref, kc_ref, kp_ref, ko_ref),
                                         (gv_ref, vc_ref, vp_ref, vo_ref)):
        g = g_ref[...]
        o_ref[...] = g * c_ref[...] + (1.0 - g) * p_ref[...]


def _merge(gk_f, gv_f, K_curr, K_prev, V_curr, V_prev, tm=256):
    b, m, d = K_curr.shape
    big = pl.BlockSpec((1, tm, d), lambda i, mt: (i, mt, 0))
    gsp = pl.BlockSpec((1, tm, 1), lambda i, mt: (i, mt, 0))
    out = jax.ShapeDtypeStruct((b, m, d), jnp.float32)
    return pl.pallas_call(
        _merge_body,
        grid=(b, m // tm),
        in_specs=[gsp, gsp, big, big, big, big],
        out_specs=[big, big],
        out_shape=[out, out],
        compiler_params=pltpu.CompilerParams(
            dimension_semantics=("parallel", "parallel")),
    )(gk_f, gv_f, K_curr, K_prev, V_curr, V_prev)


def kernel(K_curr, V_curr, K_prev, V_prev, h, momentum, active_idx,
           Wk, bk, Wv, bv, logit_eta, surprise_logit_alpha):
    b, m, d = K_curr.shape
    ng = active_idx.shape[1]

    idx3 = active_idx.astype(jnp.int32).reshape(b, 1, ng)
    q3, mult_f = _prep(h, idx3, m)
    ks, vs = _surprise(mult_f, q3, K_curr, V_curr)
    nm3, gk_f, gv_f = _gates(
        ks, vs, momentum.reshape(b, 1, 1), idx3, Wk, bk.reshape(1, ng),
        Wv, bv.reshape(1, ng), jnp.reshape(logit_eta, (1, 1)),
        jnp.reshape(surprise_logit_alpha, (1, 1)), m)
    K_out, V_out = _merge(gk_f, gv_f, K_curr, K_prev, V_curr, V_prev)
    return (K_out, V_out, nm3.reshape(b, 1))


# speedup vs baseline: 11.8177x; 1.2167x over previous
import functools

import jax
import jax.numpy as jnp
from jax.experimental import pallas as pl
from jax.experimental.pallas import tpu as pltpu


_NEG = -1e30


def _prep_body(h_ref, idx_ref, q_ref, mult_ref, *, cm):
    q_ref[...] = jnp.mean(h_ref[...], axis=1, keepdims=True)
    idx = idx_ref[0]
    m = mult_ref.shape[1]
    for c in range(m // cm):
        m_ids = c * cm + jax.lax.broadcasted_iota(jnp.int32, (cm, 1), 0)
        sel = (idx == m_ids)
        mult_ref[0, pl.ds(c * cm, cm), :] = jnp.sum(
            jnp.where(sel, 1.0, 0.0), axis=1, keepdims=True)


def _prep(h, idx3, m, cm=512):
    b, seq, d = h.shape
    ng = idx3.shape[-1]
    return pl.pallas_call(
        functools.partial(_prep_body, cm=cm),
        grid=(b,),
        in_specs=[
            pl.BlockSpec((1, seq, d), lambda i: (i, 0, 0)),
            pl.BlockSpec((1, 1, ng), lambda i: (i, 0, 0)),
        ],
        out_specs=[
            pl.BlockSpec((1, 1, d), lambda i: (i, 0, 0)),
            pl.BlockSpec((1, m, 1), lambda i: (i, 0, 0)),
        ],
        out_shape=[
            jax.ShapeDtypeStruct((b, 1, d), jnp.float32),
            jax.ShapeDtypeStruct((b, m, 1), jnp.float32),
        ],
        compiler_params=pltpu.CompilerParams(
            dimension_semantics=("arbitrary",)),
    )(h, idx3)


def _surprise_body(mult_ref, q_ref, kc_ref, vc_ref, ks_ref, vs_ref, *, scale):
    q = q_ref[0]
    mult = mult_ref[0]
    has = mult > 0.0
    for (c_ref, s_ref) in ((kc_ref, ks_ref), (vc_ref, vs_ref)):
        rows = c_ref[0]
        l = jax.lax.dot_general(rows, q, (((1,), (1,)), ((), ()))) * scale
        lmax = jnp.max(jnp.where(has, l, _NEG))
        e = jnp.exp(l - lmax) * mult
        den = jnp.sum(e)
        num = jax.lax.dot_general(e, rows, (((0,), (0,)), ((), ())))
        pred = num / den
        s_ref[...] = jnp.mean((pred - q) ** 2)[None, None, None]


def _surprise(mult_f, q3, K_curr, V_curr):
    b, m, d = K_curr.shape
    scale = float(d) ** -0.5
    sca = jax.ShapeDtypeStruct((b, 1, 1), jnp.float32)
    sspec = pl.BlockSpec((1, 1, 1), lambda i: (i, 0, 0))
    big = pl.BlockSpec((1, m, d), lambda i: (i, 0, 0))
    return pl.pallas_call(
        functools.partial(_surprise_body, scale=scale),
        grid=(b,),
        in_specs=[
            pl.BlockSpec((1, m, 1), lambda i: (i, 0, 0)),
            pl.BlockSpec((1, 1, d), lambda i: (i, 0, 0)),
            big, big,
        ],
        out_specs=[sspec, sspec],
        out_shape=[sca, sca],
        compiler_params=pltpu.CompilerParams(
            dimension_semantics=("arbitrary",)),
    )(mult_f, q3, K_curr, V_curr)


def _merge_body(ks_ref, vs_ref, mom_ref, idx_ref, wk_ref, bk_ref, wv_ref,
                bv_ref, eta_ref, alpha_ref, kc_ref, kp_ref, vc_ref, vp_ref,
                nm_ref, ko_ref, vo_ref, *, tm):
    mt = pl.program_id(1)
    ks = ks_ref[...]
    vs = vs_ref[...]
    alpha = jax.nn.sigmoid(alpha_ref[0, 0])
    comb = alpha * ks + (1.0 - alpha) * vs
    eta = jax.nn.sigmoid(eta_ref[0, 0])
    nm = eta * mom_ref[...] + (1.0 - eta) * comb

    @pl.when(mt == 0)
    def _():
        nm_ref[...] = nm

    idx = idx_ref[0]
    ng = idx.shape[-1]
    m_ids = mt * tm + jax.lax.broadcasted_iota(jnp.int32, (tm, 1), 0)
    sel = (idx == m_ids)
    n_iota = jax.lax.broadcasted_iota(jnp.int32, (tm, ng), 1)
    n_sel = jnp.max(jnp.where(sel, n_iota, -1), axis=1, keepdims=True)
    pick = sel & (n_iota == n_sel)
    active = n_sel >= 0

    for (s, w_ref, b_ref, c_ref, p_ref, o_ref) in (
            (ks, wk_ref, bk_ref, kc_ref, kp_ref, ko_ref),
            (vs, wv_ref, bv_ref, vc_ref, vp_ref, vo_ref)):
        gate = jax.nn.sigmoid(s[0] * w_ref[:, 0:1].T + nm[0] * w_ref[:, 1:2].T
                              + b_ref[...])
        g = jnp.sum(jnp.where(pick, gate, 0.0), axis=1, keepdims=True)
        g = jnp.where(active, g, 1.0)[None]
        o_ref[...] = g * c_ref[...] + (1.0 - g) * p_ref[...]


def _merge(ks, vs, mom3, idx3, Wk, bk2, Wv, bv2, eta2, alpha2,
           K_curr, K_prev, V_curr, V_prev, tm=256):
    b, m, d = K_curr.shape
    ng = idx3.shape[-1]
    sspec = pl.BlockSpec((1, 1, 1), lambda i, mt: (i, 0, 0))
    big = pl.BlockSpec((1, tm, d), lambda i, mt: (i, mt, 0))
    whole = lambda shape: pl.BlockSpec(
        shape, lambda i, mt, _s=shape: tuple(0 for _ in _s))
    out = jax.ShapeDtypeStruct((b, m, d), jnp.float32)
    return pl.pallas_call(
        functools.partial(_merge_body, tm=tm),
        grid=(b, m // tm),
        in_specs=[
            sspec, sspec, sspec,
            pl.BlockSpec((1, 1, ng), lambda i, mt: (i, 0, 0)),
            whole((ng, 2)), whole((1, ng)), whole((ng, 2)), whole((1, ng)),
            whole((1, 1)), whole((1, 1)),
            big, big, big, big,
        ],
        out_specs=[sspec, big, big],
        out_shape=[jax.ShapeDtypeStruct((b, 1, 1), jnp.float32), out, out],
        compiler_params=pltpu.CompilerParams(
            dimension_semantics=("parallel", "arbitrary")),
    )(ks, vs, mom3, idx3, Wk, bk2, Wv, bv2, eta2, alpha2,
      K_curr, K_prev, V_curr, V_prev)


def kernel(K_curr, V_curr, K_prev, V_prev, h, momentum, active_idx,
           Wk, bk, Wv, bv, logit_eta, surprise_logit_alpha):
    b, m, d = K_curr.shape
    ng = active_idx.shape[1]

    idx3 = active_idx.astype(jnp.int32).reshape(b, 1, ng)
    q3, mult_f = _prep(h, idx3, m)
    ks, vs = _surprise(mult_f, q3, K_curr, V_curr)
    nm3, K_out, V_out = _merge(
        ks, vs, momentum.reshape(b, 1, 1), idx3, Wk, bk.reshape(1, ng),
        Wv, bv.reshape(1, ng), jnp.reshape(logit_eta, (1, 1)),
        jnp.reshape(surprise_logit_alpha, (1, 1)),
        K_curr, K_prev, V_curr, V_prev)
    return (K_out, V_out, nm3.reshape(b, 1))


# speedup vs baseline: 12.2397x; 1.0357x over previous
import functools

import jax
import jax.numpy as jnp
from jax.experimental import pallas as pl
from jax.experimental.pallas import tpu as pltpu


_NEG = -1e30


def _surprise_body(h_ref, idx_ref, kc_ref, vc_ref, ks_ref, vs_ref, q_ref,
                   *, cm, scale):
    q_ref[...] = jnp.mean(h_ref[...], axis=1, keepdims=True)
    q = q_ref[0]
    idx = idx_ref[0]
    m = kc_ref.shape[1]
    mults = []
    for c in range(m // cm):
        m_ids = c * cm + jax.lax.broadcasted_iota(jnp.int32, (cm, 1), 0)
        sel = (idx == m_ids)
        mults.append(jnp.sum(jnp.where(sel, 1.0, 0.0), axis=1,
                             keepdims=True))
    mult = jnp.concatenate(mults, axis=0)
    has = mult > 0.0
    for (c_ref, s_ref) in ((kc_ref, ks_ref), (vc_ref, vs_ref)):
        rows = c_ref[0]
        l = jax.lax.dot_general(rows, q, (((1,), (1,)), ((), ()))) * scale
        lmax = jnp.max(jnp.where(has, l, _NEG))
        e = jnp.exp(l - lmax) * mult
        den = jnp.sum(e)
        num = jax.lax.dot_general(e, rows, (((0,), (0,)), ((), ())))
        pred = num / den
        s_ref[...] = jnp.mean((pred - q) ** 2)[None, None, None]


def _surprise(h, idx3, K_curr, V_curr, cm=512):
    b, m, d = K_curr.shape
    seq = h.shape[1]
    ng = idx3.shape[-1]
    scale = float(d) ** -0.5
    sca = jax.ShapeDtypeStruct((b, 1, 1), jnp.float32)
    sspec = pl.BlockSpec((1, 1, 1), lambda i: (i, 0, 0))
    big = pl.BlockSpec((1, m, d), lambda i: (i, 0, 0))
    return pl.pallas_call(
        functools.partial(_surprise_body, cm=cm, scale=scale),
        grid=(b,),
        in_specs=[
            pl.BlockSpec((1, seq, d), lambda i: (i, 0, 0)),
            pl.BlockSpec((1, 1, ng), lambda i: (i, 0, 0)),
            big, big,
        ],
        out_specs=[sspec, sspec, pl.BlockSpec((1, 1, d), lambda i: (i, 0, 0))],
        out_shape=[sca, sca, jax.ShapeDtypeStruct((b, 1, d), jnp.float32)],
        compiler_params=pltpu.CompilerParams(
            dimension_semantics=("arbitrary",)),
    )(h, idx3, K_curr, V_curr)


def _merge_body(ks_ref, vs_ref, mom_ref, idx_ref, wk_ref, bk_ref, wv_ref,
                bv_ref, eta_ref, alpha_ref, kc_ref, kp_ref, vc_ref, vp_ref,
                nm_ref, ko_ref, vo_ref, *, tm):
    mt = pl.program_id(1)
    ks = ks_ref[...]
    vs = vs_ref[...]
    alpha = jax.nn.sigmoid(alpha_ref[0, 0])
    comb = alpha * ks + (1.0 - alpha) * vs
    eta = jax.nn.sigmoid(eta_ref[0, 0])
    nm = eta * mom_ref[...] + (1.0 - eta) * comb

    @pl.when(mt == 0)
    def _():
        nm_ref[...] = nm

    idx = idx_ref[0]
    ng = idx.shape[-1]
    m_ids = mt * tm + jax.lax.broadcasted_iota(jnp.int32, (tm, 1), 0)
    sel = (idx == m_ids)
    n_iota = jax.lax.broadcasted_iota(jnp.int32, (tm, ng), 1)
    n_sel = jnp.max(jnp.where(sel, n_iota, -1), axis=1, keepdims=True)
    pick = sel & (n_iota == n_sel)
    active = n_sel >= 0

    for (s, w_ref, b_ref, c_ref, p_ref, o_ref) in (
            (ks, wk_ref, bk_ref, kc_ref, kp_ref, ko_ref),
            (vs, wv_ref, bv_ref, vc_ref, vp_ref, vo_ref)):
        gate = jax.nn.sigmoid(s[0] * w_ref[:, 0:1].T + nm[0] * w_ref[:, 1:2].T
                              + b_ref[...])
        g = jnp.sum(jnp.where(pick, gate, 0.0), axis=1, keepdims=True)
        g = jnp.where(active, g, 1.0)[None]
        o_ref[...] = g * c_ref[...] + (1.0 - g) * p_ref[...]


def _merge(ks, vs, mom3, idx3, Wk, bk2, Wv, bv2, eta2, alpha2,
           K_curr, K_prev, V_curr, V_prev, tm=256):
    b, m, d = K_curr.shape
    ng = idx3.shape[-1]
    sspec = pl.BlockSpec((1, 1, 1), lambda i, mt: (i, 0, 0))
    big = pl.BlockSpec((1, tm, d), lambda i, mt: (i, mt, 0))
    whole = lambda shape: pl.BlockSpec(
        shape, lambda i, mt, _s=shape: tuple(0 for _ in _s))
    out = jax.ShapeDtypeStruct((b, m, d), jnp.float32)
    return pl.pallas_call(
        functools.partial(_merge_body, tm=tm),
        grid=(b, m // tm),
        in_specs=[
            sspec, sspec, sspec,
            pl.BlockSpec((1, 1, ng), lambda i, mt: (i, 0, 0)),
            whole((ng, 2)), whole((1, ng)), whole((ng, 2)), whole((1, ng)),
            whole((1, 1)), whole((1, 1)),
            big, big, big, big,
        ],
        out_specs=[sspec, big, big],
        out_shape=[jax.ShapeDtypeStruct((b, 1, 1), jnp.float32), out, out],
        compiler_params=pltpu.CompilerParams(
            dimension_semantics=("parallel", "arbitrary")),
    )(ks, vs, mom3, idx3, Wk, bk2, Wv, bv2, eta2, alpha2,
      K_curr, K_prev, V_curr, V_prev)


def kernel(K_curr, V_curr, K_prev, V_prev, h, momentum, active_idx,
           Wk, bk, Wv, bv, logit_eta, surprise_logit_alpha):
    b, m, d = K_curr.shape
    ng = active_idx.shape[1]

    idx3 = active_idx.astype(jnp.int32).reshape(b, 1, ng)
    ks, vs, _ = _surprise(h, idx3, K_curr, V_curr)
    nm3, K_out, V_out = _merge(
        ks, vs, momentum.reshape(b, 1, 1), idx3, Wk, bk.reshape(1, ng),
        Wv, bv.reshape(1, ng), jnp.reshape(logit_eta, (1, 1)),
        jnp.reshape(surprise_logit_alpha, (1, 1)),
        K_curr, K_prev, V_curr, V_prev)
    return (K_out, V_out, nm3.reshape(b, 1))
